# baseline (speedup 1.0000x reference)
import functools

import jax
import jax.numpy as jnp
from jax import lax
from jax.experimental import pallas as pl
from jax.experimental.pallas import tpu as pltpu
from jax.experimental.pallas import tpu_sc as plsc

NUM_EXPERTS = 8
BATCH = 4096
D_IN = 2048
D_OUT = 2048
TILE_M = 128
PADDED = BATCH + NUM_EXPERTS * TILE_M
NUM_TILES = PADDED // TILE_M

_NC, _NS = 2, 16
_NW = _NC * _NS
_CH = 16
_NBUF = 3


@functools.lru_cache(maxsize=None)
def _sc_mesh():
    return plsc.VectorSubcoreMesh(
        core_axis_name="c", subcore_axis_name="s", num_cores=_NC, num_subcores=_NS
    )


def _routing(domains):
    d = domains.astype(jnp.int32)
    onehot = (d[:, None] == jnp.arange(NUM_EXPERTS, dtype=jnp.int32)[None, :])
    oh = onehot.astype(jnp.float32)
    ohb = oh.reshape(32, 128, NUM_EXPERTS)
    tri = jnp.tril(jnp.ones((128, 128), jnp.float32))
    intra = jnp.einsum("ij,bjk->bik", tri, ohb,
                       preferred_element_type=jnp.float32)
    blocksum = jnp.sum(ohb, axis=1)
    blockpre = jnp.cumsum(blocksum, axis=0) - blocksum
    cum_incl = (intra + blockpre[:, None, :]).reshape(BATCH, NUM_EXPERTS)
    rank = jnp.sum(cum_incl * oh, axis=1).astype(jnp.int32) - 1
    counts = jnp.sum(blocksum, axis=0).astype(jnp.int32)
    padded_counts = ((counts + TILE_M - 1) // TILE_M) * TILE_M
    ends = jnp.cumsum(padded_counts)
    starts = ends - padded_counts
    dest = starts[d] + rank
    seg = (starts // TILE_M).astype(jnp.int32)
    nseg = (padded_counts // TILE_M).astype(jnp.int32)
    return dest, seg, nseg


@functools.lru_cache(maxsize=None)
def _make_sc_scatter(D):
    rpw = BATCH // _NW
    nch = rpw // _CH

    @functools.partial(
        pl.kernel,
        out_type=jax.ShapeDtypeStruct((PADDED, D), jnp.float32),
        mesh=_sc_mesh(),
        scratch_types=[
            pltpu.VMEM((nch, _CH), jnp.int32),
            [pltpu.VMEM((_CH, D), jnp.float32) for _ in range(_NBUF)],
            [pltpu.SemaphoreType.DMA for _ in range(_NBUF)],
            [pltpu.SemaphoreType.DMA for _ in range(_NBUF)],
        ],
    )
    def scatter_k(rows_hbm, idx_hbm, out_hbm, idx_v, bufs, rsems, wsems):
        wid = lax.axis_index("s") * _NC + lax.axis_index("c")
        base = wid * rpw
        pltpu.sync_copy(idx_hbm.at[wid], idx_v)
        reads = [None] * nch
        writes = [None] * nch

        def start_read(c):
            reads[c] = pltpu.async_copy(
                rows_hbm.at[pl.ds(base + c * _CH, _CH)],
                bufs[c % _NBUF],
                rsems[c % _NBUF],
            )

        for c in range(min(_NBUF, nch)):
            start_read(c)
        for c in range(nch):
            reads[c].wait()
            writes[c] = pltpu.async_copy(
                bufs[c % _NBUF], out_hbm.at[idx_v.at[c]], wsems[c % _NBUF]
            )
            if c + _NBUF < nch:
                writes[c].wait()
                start_read(c + _NBUF)
        for c in range(max(0, nch - _NBUF), nch):
            writes[c].wait()

    return scatter_k


@functools.lru_cache(maxsize=None)
def _make_sc_gather(B, D):
    rpw = B // _NW
    nch = rpw // _CH

    @functools.partial(
        pl.kernel,
        out_type=jax.ShapeDtypeStruct((B, D), jnp.float32),
        mesh=_sc_mesh(),
        scratch_types=[
            pltpu.VMEM((rpw,), jnp.int32),
            [pltpu.VMEM((_CH, D), jnp.float32) for _ in range(_NBUF)],
            [pltpu.SemaphoreType.DMA for _ in range(_NBUF)],
        ],
    )
    def gather_k(table_hbm, idx_hbm, out_hbm, idx_v, bufs, sems):
        wid = lax.axis_index("s") * _NC + lax.axis_index("c")
        base = wid * rpw
        pltpu.sync_copy(idx_hbm.at[pl.ds(base, rpw)], idx_v)
        copies = [None] * nch

        def start(c):
            copies[c] = pltpu.async_copy(
                table_hbm.at[idx_v.at[pl.ds(c * _CH, _CH)]],
                bufs[c % _NBUF],
                sems[c % _NBUF],
            )

        for c in range(min(_NBUF, nch)):
            start(c)
        for c in range(nch):
            copies[c].wait()
            pltpu.sync_copy(bufs[c % _NBUF], out_hbm.at[pl.ds(base + c * _CH, _CH)])
            if c + _NBUF < nch:
                start(c + _NBUF)

    return gather_k


_MAX_T = BATCH // TILE_M


def _mm_body(seg_ref, nseg_ref, x_hbm, w_ref, b_ref, y_hbm, xbufs, ybufs,
             xsems, ysems):
    e = pl.program_id(0)
    t0 = seg_ref[e]
    nt = nseg_ref[e]

    def xcopy(t, s):
        return pltpu.make_async_copy(
            x_hbm.at[pl.ds((t0 + t) * TILE_M, TILE_M), :], xbufs[s], xsems[s]
        )

    def ycopy(t, s):
        return pltpu.make_async_copy(
            ybufs[s], y_hbm.at[pl.ds((t0 + t) * TILE_M, TILE_M), :], ysems[s]
        )

    for t in range(2):
        @pl.when(t < nt)
        def _prime():
            xcopy(t, t).start()

    for t in range(_MAX_T):
        s = t % 2

        @pl.when(t < nt)
        def _tile():
            xcopy(t, s).wait()
            if t >= 2:
                ycopy(t - 2, s).wait()
            ybufs[s][...] = (
                jnp.dot(xbufs[s][...], w_ref[0],
                        preferred_element_type=jnp.float32)
                + b_ref[0]
            )
            ycopy(t, s).start()
            if t + 2 < _MAX_T:
                @pl.when(t + 2 < nt)
                def _next():
                    xcopy(t + 2, s).start()

    for s in range(2):
        @pl.when(nt > s)
        def _drain():
            ycopy(0, s).wait()


def _grouped_matmul(x_sorted, W, b, seg, nseg):
    grid_spec = pltpu.PrefetchScalarGridSpec(
        num_scalar_prefetch=2,
        grid=(NUM_EXPERTS,),
        in_specs=[
            pl.BlockSpec(memory_space=pl.ANY),
            pl.BlockSpec((1, D_IN, D_OUT), lambda e, seg, nseg: (e, 0, 0)),
            pl.BlockSpec((1, 1, D_OUT), lambda e, seg, nseg: (e, 0, 0)),
        ],
        out_specs=pl.BlockSpec(memory_space=pl.ANY),
        scratch_shapes=[
            [pltpu.VMEM((TILE_M, D_IN), jnp.float32) for _ in range(2)],
            [pltpu.VMEM((TILE_M, D_OUT), jnp.float32) for _ in range(2)],
            [pltpu.SemaphoreType.DMA for _ in range(2)],
            [pltpu.SemaphoreType.DMA for _ in range(2)],
        ],
    )
    return pl.pallas_call(
        _mm_body,
        grid_spec=grid_spec,
        out_shape=jax.ShapeDtypeStruct((PADDED, D_OUT), jnp.float32),
    )(seg, nseg, x_sorted, W, b.reshape(NUM_EXPERTS, 1, D_OUT))


def kernel(images, domains, W, b):
    dest, seg, nseg = _routing(domains)
    idx3 = dest.reshape(_NW, BATCH // _NW // _CH, _CH)
    x_sorted = _make_sc_scatter(D_IN)(images, idx3)
    y_sorted = _grouped_matmul(x_sorted, W, b, seg, nseg)
    outputs = _make_sc_gather(BATCH, D_OUT)(y_sorted, dest)
    return outputs


# speedup vs baseline: 2.0313x; 2.0313x over previous
import functools

import jax
import jax.numpy as jnp
from jax import lax
from jax.experimental import pallas as pl
from jax.experimental.pallas import tpu as pltpu
from jax.experimental.pallas import tpu_sc as plsc

NUM_EXPERTS = 8
BATCH = 4096
D_IN = 2048
D_OUT = 2048
TILE_M = 128
PADDED = BATCH + NUM_EXPERTS * TILE_M
NUM_TILES = PADDED // TILE_M

_NC, _NS = 2, 16
_NW = _NC * _NS
_CH = 16
_NBUF = 3


@functools.lru_cache(maxsize=None)
def _sc_mesh():
    return plsc.VectorSubcoreMesh(
        core_axis_name="c", subcore_axis_name="s", num_cores=_NC, num_subcores=_NS
    )


def _routing(domains):
    d = domains.astype(jnp.int32)
    onehot = (d[:, None] == jnp.arange(NUM_EXPERTS, dtype=jnp.int32)[None, :])
    oh = onehot.astype(jnp.float32)
    ohb = oh.reshape(32, 128, NUM_EXPERTS)
    tri = jnp.tril(jnp.ones((128, 128), jnp.float32))
    intra = jnp.einsum("ij,bjk->bik", tri, ohb,
                       preferred_element_type=jnp.float32)
    blocksum = jnp.sum(ohb, axis=1)
    blockpre = jnp.cumsum(blocksum, axis=0) - blocksum
    cum_incl = (intra + blockpre[:, None, :]).reshape(BATCH, NUM_EXPERTS)
    rank = jnp.sum(cum_incl * oh, axis=1).astype(jnp.int32) - 1
    counts = jnp.sum(blocksum, axis=0).astype(jnp.int32)
    padded_counts = ((counts + TILE_M - 1) // TILE_M) * TILE_M
    ends = jnp.cumsum(padded_counts)
    starts = ends - padded_counts
    dest = starts[d] + rank
    tile_ids = jnp.arange(NUM_TILES, dtype=jnp.int32) * TILE_M
    tile_expert = jnp.minimum(
        jnp.sum((ends[None, :] <= tile_ids[:, None]).astype(jnp.int32), axis=1),
        NUM_EXPERTS - 1,
    ).astype(jnp.int32)
    return dest, tile_expert


@functools.lru_cache(maxsize=None)
def _make_sc_scatter(D):
    rpw = BATCH // _NW
    nch = rpw // _CH

    @functools.partial(
        pl.kernel,
        out_type=jax.ShapeDtypeStruct((PADDED, D), jnp.float32),
        mesh=_sc_mesh(),
        scratch_types=[
            pltpu.VMEM((nch, _CH), jnp.int32),
            [pltpu.VMEM((_CH, D), jnp.float32) for _ in range(_NBUF)],
            [pltpu.SemaphoreType.DMA for _ in range(_NBUF)],
            [pltpu.SemaphoreType.DMA for _ in range(_NBUF)],
        ],
    )
    def scatter_k(rows_hbm, idx_hbm, out_hbm, idx_v, bufs, rsems, wsems):
        wid = lax.axis_index("s") * _NC + lax.axis_index("c")
        base = wid * rpw
        pltpu.sync_copy(idx_hbm.at[wid], idx_v)
        reads = [None] * nch
        writes = [None] * nch

        def start_read(c):
            reads[c] = pltpu.async_copy(
                rows_hbm.at[pl.ds(base + c * _CH, _CH)],
                bufs[c % _NBUF],
                rsems[c % _NBUF],
            )

        for c in range(min(_NBUF, nch)):
            start_read(c)
        for c in range(nch):
            reads[c].wait()
            writes[c] = pltpu.async_copy(
                bufs[c % _NBUF], out_hbm.at[idx_v.at[c]], wsems[c % _NBUF]
            )
            if c + _NBUF < nch:
                writes[c].wait()
                start_read(c + _NBUF)
        for c in range(max(0, nch - _NBUF), nch):
            writes[c].wait()

    return scatter_k


@functools.lru_cache(maxsize=None)
def _make_sc_gather(B, D):
    rpw = B // _NW
    nch = rpw // _CH

    @functools.partial(
        pl.kernel,
        out_type=jax.ShapeDtypeStruct((B, D), jnp.float32),
        mesh=_sc_mesh(),
        scratch_types=[
            pltpu.VMEM((rpw,), jnp.int32),
            [pltpu.VMEM((_CH, D), jnp.float32) for _ in range(_NBUF)],
            [pltpu.SemaphoreType.DMA for _ in range(_NBUF)],
        ],
    )
    def gather_k(table_hbm, idx_hbm, out_hbm, idx_v, bufs, sems):
        wid = lax.axis_index("s") * _NC + lax.axis_index("c")
        base = wid * rpw
        pltpu.sync_copy(idx_hbm.at[pl.ds(base, rpw)], idx_v)
        copies = [None] * nch

        def start(c):
            copies[c] = pltpu.async_copy(
                table_hbm.at[idx_v.at[pl.ds(c * _CH, _CH)]],
                bufs[c % _NBUF],
                sems[c % _NBUF],
            )

        for c in range(min(_NBUF, nch)):
            start(c)
        for c in range(nch):
            copies[c].wait()
            pltpu.sync_copy(bufs[c % _NBUF], out_hbm.at[pl.ds(base + c * _CH, _CH)])
            if c + _NBUF < nch:
                start(c + _NBUF)

    return gather_k


def _mm_body(te_ref, x_ref, w_ref, b_ref, y_ref):
    del te_ref
    y_ref[...] = (
        jnp.dot(x_ref[...], w_ref[0], preferred_element_type=jnp.float32)
        + b_ref[0]
    )


def _grouped_matmul(x_sorted, W, b, tile_expert):
    grid_spec = pltpu.PrefetchScalarGridSpec(
        num_scalar_prefetch=1,
        grid=(NUM_TILES,),
        in_specs=[
            pl.BlockSpec((TILE_M, D_IN), lambda i, te: (i, 0)),
            pl.BlockSpec((1, D_IN, D_OUT), lambda i, te: (te[i], 0, 0)),
            pl.BlockSpec((1, 1, D_OUT), lambda i, te: (te[i], 0, 0)),
        ],
        out_specs=pl.BlockSpec((TILE_M, D_OUT), lambda i, te: (i, 0)),
    )
    return pl.pallas_call(
        _mm_body,
        grid_spec=grid_spec,
        out_shape=jax.ShapeDtypeStruct((PADDED, D_OUT), jnp.float32),
    )(tile_expert, x_sorted, W, b.reshape(NUM_EXPERTS, 1, D_OUT))


def kernel(images, domains, W, b):
    dest, tile_expert = _routing(domains)
    idx3 = dest.reshape(_NW, BATCH // _NW // _CH, _CH)
    x_sorted = _make_sc_scatter(D_IN)(images, idx3)
    y_sorted = _grouped_matmul(x_sorted, W, b, tile_expert)
    outputs = _make_sc_gather(BATCH, D_OUT)(y_sorted, dest)
    return outputs
